# baseline (speedup 1.0000x reference)
import functools

import jax
import jax.numpy as jnp
from jax import lax
from jax.experimental import pallas as pl
from jax.experimental.pallas import tpu as pltpu
from jax.experimental.pallas import tpu_sc as plsc

D_MODEL = 64
D_PAD = 128
NC = 2
NS = 16
NW = NC * NS
CHUNK = 128
NBUF = 5
V = 1000000

RB = 128
BLK_WORDS = RB * D_PAD
NBLK_FULL = V // RB
TAIL = V - NBLK_FULL * RB


@functools.lru_cache(maxsize=None)
def _build_transpose():
    mesh = plsc.VectorSubcoreMesh(core_axis_name="c", subcore_axis_name="s")

    @functools.partial(
        pl.kernel,
        mesh=mesh,
        out_type=jax.ShapeDtypeStruct((V * D_PAD,), jnp.float32),
        scratch_types=[
            pltpu.VMEM((2, 64, RB), jnp.float32),
            pltpu.VMEM((2, BLK_WORDS), jnp.float32),
            pltpu.SemaphoreType.DMA((2,)),
            pltpu.SemaphoreType.DMA((2,)),
        ],
        compiler_params=pltpu.CompilerParams(
            use_tc_tiling_on_sc=True, needs_layout_passes=False),
    )
    def tkern(wt_hbm, out_hbm, tiles_v, blk_v, sem_r, sem_w):
        wid = lax.axis_index("s") * NC + lax.axis_index("c")
        per_w = NBLK_FULL // NW
        rem = NBLK_FULL - per_w * NW
        start = wid * per_w + jnp.minimum(wid, rem)
        cnt = per_w + jnp.where(wid < rem, 1, 0)

        iota = lax.iota(jnp.int32, 16)
        lane_bases = [(iota + 16 * grp) * D_PAD for grp in range(8)]

        def rearrange(slot):
            slot_vec = jnp.zeros((16,), jnp.int32) + slot
            for d in range(64):
                for grp in range(8):
                    vals = tiles_v[slot, d, pl.ds(16 * grp, 16)]
                    plsc.store_scatter(
                        blk_v, [slot_vec, lane_bases[grp] + d], vals)

        def read_block(k, slot):
            r0 = k * RB
            for a in range(8):
                pltpu.async_copy(
                    wt_hbm.at[a, :, pl.ds(r0, RB)],
                    tiles_v.at[slot, pl.ds(a * 8, 8)], sem_r.at[slot])

        def wait_read(k, slot):
            r0 = k * RB
            for a in range(8):
                pltpu.make_async_copy(
                    wt_hbm.at[a, :, pl.ds(r0, RB)],
                    tiles_v.at[slot, pl.ds(a * 8, 8)], sem_r.at[slot]).wait()

        def write_block(k, slot):
            pltpu.async_copy(
                blk_v.at[slot], out_hbm.at[pl.ds(k * BLK_WORDS, BLK_WORDS)],
                sem_w.at[slot])

        def wait_write(k, slot):
            pltpu.make_async_copy(
                blk_v.at[slot], out_hbm.at[pl.ds(k * BLK_WORDS, BLK_WORDS)],
                sem_w.at[slot]).wait()

        read_block(start, 0)

        def body(i, carry):
            k = start + i
            for slot in range(2):
                @pl.when(lax.rem(i, 2) == slot)
                def _():
                    @pl.when(i + 1 < cnt)
                    def _():
                        read_block(k + 1, 1 - slot)
                    wait_read(k, slot)

                    @pl.when(i >= 2)
                    def _():
                        wait_write(k - 2, slot)
                    rearrange(slot)
                    write_block(k, slot)
            return carry

        lax.fori_loop(0, cnt, body, 0)
        for slot in range(2):
            @pl.when(cnt > slot)
            def _():
                i_last = cnt - 1 - lax.rem(cnt - 1 - slot, 2)
                wait_write(start + i_last, slot)

        @pl.when(wid == NW - 1)
        def _():
            r0 = jnp.int32(NBLK_FULL) * jnp.int32(RB)
            for a in range(8):
                pltpu.async_copy(
                    wt_hbm.at[a, :, pl.ds(r0, RB)],
                    tiles_v.at[0, pl.ds(a * 8, 8)], sem_r.at[0])
            for a in range(8):
                pltpu.make_async_copy(
                    wt_hbm.at[a, :, pl.ds(r0, RB)],
                    tiles_v.at[0, pl.ds(a * 8, 8)], sem_r.at[0]).wait()
            rearrange(0)
            pltpu.sync_copy(
                blk_v.at[0, pl.ds(0, TAIL * D_PAD)],
                out_hbm.at[pl.ds(r0 * jnp.int32(D_PAD), TAIL * D_PAD)])

    return tkern


@functools.lru_cache(maxsize=None)
def _build_gather(b_total):
    b_per_w = b_total // NW
    n_chunks = b_per_w // CHUNK
    n_groups = n_chunks // NBUF
    mesh = plsc.VectorSubcoreMesh(core_axis_name="c", subcore_axis_name="s")

    @functools.partial(
        pl.kernel,
        mesh=mesh,
        out_type=jax.ShapeDtypeStruct((b_total, D_PAD), jnp.float32),
        scratch_types=[
            pltpu.VMEM((n_chunks, CHUNK), jnp.int32),
            pltpu.VMEM((NBUF, CHUNK, D_PAD), jnp.float32),
            pltpu.SemaphoreType.DMA((NBUF,)),
            pltpu.SemaphoreType.DMA((NBUF,)),
        ],
        compiler_params=pltpu.CompilerParams(use_tc_tiling_on_sc=True),
    )
    def embed(idx_hbm, table_hbm, out_hbm, idx_v, rows_v, sem_g, sem_s):
        wid = lax.axis_index("s") * NC + lax.axis_index("c")
        base = wid * b_per_w
        pltpu.sync_copy(idx_hbm.at[wid], idx_v)
        for b in range(NBUF):
            pltpu.async_copy(table_hbm.at[idx_v.at[b]], rows_v.at[b], sem_g.at[b])

        def group(g, carry):
            cbase = g * NBUF
            for b in range(NBUF):
                c = cbase + b
                pltpu.make_async_copy(
                    table_hbm.at[idx_v.at[c]], rows_v.at[b], sem_g.at[b]).wait()
                pltpu.async_copy(
                    rows_v.at[b],
                    out_hbm.at[pl.ds(base + c * CHUNK, CHUNK)],
                    sem_s.at[b])
            for b in range(NBUF):
                c = cbase + b
                pltpu.make_async_copy(
                    rows_v.at[b],
                    out_hbm.at[pl.ds(base + c * CHUNK, CHUNK)],
                    sem_s.at[b]).wait()

                @pl.when(g + 1 < n_groups)
                def _():
                    pltpu.async_copy(
                        table_hbm.at[idx_v.at[c + NBUF]], rows_v.at[b], sem_g.at[b])
            return carry

        lax.fori_loop(0, n_groups, group, 0)

    return embed


def kernel(tokens, W_E):
    bsz, seq = tokens.shape
    b_total = bsz * seq
    idx = tokens.astype(jnp.int32).reshape(NW, b_total // NW // CHUNK, CHUNK)
    wt = W_E.T.reshape(8, 8, V)
    table = _build_transpose()(wt).reshape(V, D_PAD)
    out = _build_gather(b_total)(idx, table)
    return out[:, :D_MODEL].reshape(bsz, seq, D_MODEL)


# speedup vs baseline: 1.0136x; 1.0136x over previous
import functools

import jax
import jax.numpy as jnp
from jax import lax
from jax.experimental import pallas as pl
from jax.experimental.pallas import tpu as pltpu
from jax.experimental.pallas import tpu_sc as plsc

D_MODEL = 64
D_PAD = 128
NC = 2
NS = 16
NW = NC * NS
CHUNK = 128
NBUF = 5
V = 1000000

RB = 128
BLK_WORDS = RB * D_PAD
NBLK_FULL = V // RB
TAIL = V - NBLK_FULL * RB


@functools.lru_cache(maxsize=None)
def _build_transpose():
    mesh = plsc.VectorSubcoreMesh(core_axis_name="c", subcore_axis_name="s")

    @functools.partial(
        pl.kernel,
        mesh=mesh,
        out_type=jax.ShapeDtypeStruct((V * D_PAD,), jnp.float32),
        scratch_types=[
            pltpu.VMEM((2, 64, RB), jnp.float32),
            pltpu.VMEM((2, BLK_WORDS), jnp.float32),
            pltpu.SemaphoreType.DMA((2,)),
            pltpu.SemaphoreType.DMA((2,)),
        ],
        compiler_params=pltpu.CompilerParams(
            use_tc_tiling_on_sc=True, needs_layout_passes=False),
    )
    def tkern(wt_hbm, out_hbm, tiles_v, blk_v, sem_r, sem_w):
        wid = lax.axis_index("s") * NC + lax.axis_index("c")
        per_w = NBLK_FULL // NW
        rem = NBLK_FULL - per_w * NW
        start = wid * per_w + jnp.minimum(wid, rem)
        cnt = per_w + jnp.where(wid < rem, 1, 0)

        iota = lax.iota(jnp.int32, 16)
        lane_bases = [(iota + 16 * grp) * D_PAD for grp in range(8)]

        def rearrange(slot):
            slot_vec = jnp.zeros((16,), jnp.int32) + slot
            for d in range(64):
                vals = [tiles_v[slot, d, pl.ds(16 * grp, 16)] for grp in range(8)]
                idxs = [lane_bases[grp] + d for grp in range(8)]
                for grp in range(8):
                    plsc.store_scatter(blk_v, [slot_vec, idxs[grp]], vals[grp])

        def read_block(k, slot):
            r0 = k * RB
            for a in range(8):
                pltpu.async_copy(
                    wt_hbm.at[a, :, pl.ds(r0, RB)],
                    tiles_v.at[slot, pl.ds(a * 8, 8)], sem_r.at[slot])

        def wait_read(k, slot):
            r0 = k * RB
            for a in range(8):
                pltpu.make_async_copy(
                    wt_hbm.at[a, :, pl.ds(r0, RB)],
                    tiles_v.at[slot, pl.ds(a * 8, 8)], sem_r.at[slot]).wait()

        def write_block(k, slot):
            pltpu.async_copy(
                blk_v.at[slot], out_hbm.at[pl.ds(k * BLK_WORDS, BLK_WORDS)],
                sem_w.at[slot])

        def wait_write(k, slot):
            pltpu.make_async_copy(
                blk_v.at[slot], out_hbm.at[pl.ds(k * BLK_WORDS, BLK_WORDS)],
                sem_w.at[slot]).wait()

        read_block(start, 0)

        def body(i, carry):
            k = start + i
            for slot in range(2):
                @pl.when(lax.rem(i, 2) == slot)
                def _():
                    @pl.when(i + 1 < cnt)
                    def _():
                        read_block(k + 1, 1 - slot)
                    wait_read(k, slot)

                    @pl.when(i >= 2)
                    def _():
                        wait_write(k - 2, slot)
                    rearrange(slot)
                    write_block(k, slot)
            return carry

        lax.fori_loop(0, cnt, body, 0)
        for slot in range(2):
            @pl.when(cnt > slot)
            def _():
                i_last = cnt - 1 - lax.rem(cnt - 1 - slot, 2)
                wait_write(start + i_last, slot)

        @pl.when(wid == NW - 1)
        def _():
            r0 = jnp.int32(NBLK_FULL) * jnp.int32(RB)
            for a in range(8):
                pltpu.async_copy(
                    wt_hbm.at[a, :, pl.ds(r0, RB)],
                    tiles_v.at[0, pl.ds(a * 8, 8)], sem_r.at[0])
            for a in range(8):
                pltpu.make_async_copy(
                    wt_hbm.at[a, :, pl.ds(r0, RB)],
                    tiles_v.at[0, pl.ds(a * 8, 8)], sem_r.at[0]).wait()
            rearrange(0)
            pltpu.sync_copy(
                blk_v.at[0, pl.ds(0, TAIL * D_PAD)],
                out_hbm.at[pl.ds(r0 * jnp.int32(D_PAD), TAIL * D_PAD)])

    return tkern


@functools.lru_cache(maxsize=None)
def _build_gather(b_total):
    b_per_w = b_total // NW
    n_chunks = b_per_w // CHUNK
    n_groups = n_chunks // NBUF
    mesh = plsc.VectorSubcoreMesh(core_axis_name="c", subcore_axis_name="s")

    @functools.partial(
        pl.kernel,
        mesh=mesh,
        out_type=jax.ShapeDtypeStruct((b_total, D_PAD), jnp.float32),
        scratch_types=[
            pltpu.VMEM((n_chunks, CHUNK), jnp.int32),
            pltpu.VMEM((NBUF, CHUNK, D_PAD), jnp.float32),
            pltpu.SemaphoreType.DMA((NBUF,)),
            pltpu.SemaphoreType.DMA((NBUF,)),
        ],
        compiler_params=pltpu.CompilerParams(use_tc_tiling_on_sc=True),
    )
    def embed(idx_hbm, table_hbm, out_hbm, idx_v, rows_v, sem_g, sem_s):
        wid = lax.axis_index("s") * NC + lax.axis_index("c")
        base = wid * b_per_w
        pltpu.sync_copy(idx_hbm.at[wid], idx_v)
        for b in range(NBUF):
            pltpu.async_copy(table_hbm.at[idx_v.at[b]], rows_v.at[b], sem_g.at[b])

        def group(g, carry):
            cbase = g * NBUF
            for b in range(NBUF):
                c = cbase + b
                pltpu.make_async_copy(
                    table_hbm.at[idx_v.at[c]], rows_v.at[b], sem_g.at[b]).wait()
                pltpu.async_copy(
                    rows_v.at[b],
                    out_hbm.at[pl.ds(base + c * CHUNK, CHUNK)],
                    sem_s.at[b])
            for b in range(NBUF):
                c = cbase + b
                pltpu.make_async_copy(
                    rows_v.at[b],
                    out_hbm.at[pl.ds(base + c * CHUNK, CHUNK)],
                    sem_s.at[b]).wait()

                @pl.when(g + 1 < n_groups)
                def _():
                    pltpu.async_copy(
                        table_hbm.at[idx_v.at[c + NBUF]], rows_v.at[b], sem_g.at[b])
            return carry

        lax.fori_loop(0, n_groups, group, 0)

    return embed


def kernel(tokens, W_E):
    bsz, seq = tokens.shape
    b_total = bsz * seq
    idx = tokens.astype(jnp.int32).reshape(NW, b_total // NW // CHUNK, CHUNK)
    wt = W_E.T.reshape(8, 8, V)
    table = _build_transpose()(wt).reshape(V, D_PAD)
    out = _build_gather(b_total)(idx, table)
    return out[:, :D_MODEL].reshape(bsz, seq, D_MODEL)


# speedup vs baseline: 1.5797x; 1.5585x over previous
import functools

import jax
import jax.numpy as jnp
from jax import lax
from jax.experimental import pallas as pl
from jax.experimental.pallas import tpu as pltpu
from jax.experimental.pallas import tpu_sc as plsc

D_MODEL = 64
D_PAD = 128
NC = 2
NS = 16
NW = NC * NS
CHUNK = 128
NBUF = 5
V = 1000000
PAD_ROWS = 2000


@functools.lru_cache(maxsize=None)
def _build_pad():
    def body(in_ref, out_ref):
        out_ref[:, :D_MODEL] = in_ref[...]

    return pl.pallas_call(
        body,
        grid=(V // PAD_ROWS,),
        in_specs=[pl.BlockSpec((PAD_ROWS, D_MODEL), lambda i: (i, 0))],
        out_specs=pl.BlockSpec((PAD_ROWS, D_PAD), lambda i: (i, 0)),
        out_shape=jax.ShapeDtypeStruct((V, D_PAD), jnp.float32),
    )


@functools.lru_cache(maxsize=None)
def _build_gather(b_total):
    b_per_w = b_total // NW
    n_chunks = b_per_w // CHUNK
    n_groups = n_chunks // NBUF
    mesh = plsc.VectorSubcoreMesh(core_axis_name="c", subcore_axis_name="s")

    @functools.partial(
        pl.kernel,
        mesh=mesh,
        out_type=jax.ShapeDtypeStruct((b_total, D_PAD), jnp.float32),
        scratch_types=[
            pltpu.VMEM((n_chunks, CHUNK), jnp.int32),
            pltpu.VMEM((NBUF, CHUNK, D_PAD), jnp.float32),
            pltpu.SemaphoreType.DMA((NBUF,)),
            pltpu.SemaphoreType.DMA((NBUF,)),
        ],
        compiler_params=pltpu.CompilerParams(use_tc_tiling_on_sc=True),
    )
    def embed(idx_hbm, table_hbm, out_hbm, idx_v, rows_v, sem_g, sem_s):
        wid = lax.axis_index("s") * NC + lax.axis_index("c")
        base = wid * b_per_w
        pltpu.sync_copy(idx_hbm.at[wid], idx_v)

        for b in range(NBUF):
            pltpu.async_copy(table_hbm.at[idx_v.at[b]], rows_v.at[b], sem_g.at[b])

        def group(g, carry):
            cbase = g * NBUF
            for b in range(NBUF):
                c = cbase + b
                pltpu.make_async_copy(
                    table_hbm.at[idx_v.at[c]], rows_v.at[b], sem_g.at[b]).wait()
                pltpu.async_copy(
                    rows_v.at[b],
                    out_hbm.at[pl.ds(base + c * CHUNK, CHUNK)],
                    sem_s.at[b])
            for b in range(NBUF):
                c = cbase + b
                pltpu.make_async_copy(
                    rows_v.at[b],
                    out_hbm.at[pl.ds(base + c * CHUNK, CHUNK)],
                    sem_s.at[b]).wait()

                @pl.when(g + 1 < n_groups)
                def _():
                    pltpu.async_copy(
                        table_hbm.at[idx_v.at[c + NBUF]], rows_v.at[b], sem_g.at[b])
            return carry

        lax.fori_loop(0, n_groups, group, 0)

    return embed


def kernel(tokens, W_E):
    bsz, seq = tokens.shape
    b_total = bsz * seq
    idx = tokens.astype(jnp.int32).reshape(NW, b_total // NW // CHUNK, CHUNK)
    table = _build_pad()(W_E)
    out = _build_gather(b_total)(idx, table)
    return out[:, :D_MODEL].reshape(bsz, seq, D_MODEL)


# speedup vs baseline: 2.2806x; 1.4437x over previous
import functools

import jax
import jax.numpy as jnp
from jax import lax
from jax.experimental import pallas as pl
from jax.experimental.pallas import tpu as pltpu
from jax.experimental.pallas import tpu_sc as plsc

D_MODEL = 64
D_PAD = 128
NC = 2
NS = 16
NW = NC * NS
CHUNK = 128
NBUF = 5


@functools.lru_cache(maxsize=None)
def _build(b_total):
    b_per_w = b_total // NW
    n_chunks = b_per_w // CHUNK
    n_groups = n_chunks // NBUF
    mesh = plsc.VectorSubcoreMesh(core_axis_name="c", subcore_axis_name="s")

    @functools.partial(
        pl.kernel,
        mesh=mesh,
        out_type=jax.ShapeDtypeStruct((b_total, D_PAD), jnp.float32),
        scratch_types=[
            pltpu.VMEM((n_chunks, CHUNK), jnp.int32),
            pltpu.VMEM((NBUF, CHUNK, D_PAD), jnp.float32),
            pltpu.SemaphoreType.DMA((NBUF,)),
            pltpu.SemaphoreType.DMA((NBUF,)),
        ],
        compiler_params=pltpu.CompilerParams(use_tc_tiling_on_sc=True),
    )
    def embed(idx_hbm, table_hbm, out_hbm, idx_v, rows_v, sem_g, sem_s):
        wid = lax.axis_index("s") * NC + lax.axis_index("c")
        base = wid * b_per_w
        pltpu.sync_copy(idx_hbm.at[wid], idx_v)

        for b in range(NBUF):
            pltpu.async_copy(table_hbm.at[idx_v.at[b]], rows_v.at[b], sem_g.at[b])

        def group(g, carry):
            cbase = g * NBUF
            for b in range(NBUF):
                c = cbase + b
                pltpu.make_async_copy(
                    table_hbm.at[idx_v.at[c]], rows_v.at[b], sem_g.at[b]).wait()
                pltpu.async_copy(
                    rows_v.at[b],
                    out_hbm.at[pl.ds(base + c * CHUNK, CHUNK)],
                    sem_s.at[b])
            for b in range(NBUF):
                c = cbase + b
                pltpu.make_async_copy(
                    rows_v.at[b],
                    out_hbm.at[pl.ds(base + c * CHUNK, CHUNK)],
                    sem_s.at[b]).wait()

                @pl.when(g + 1 < n_groups)
                def _():
                    pltpu.async_copy(
                        table_hbm.at[idx_v.at[c + NBUF]], rows_v.at[b], sem_g.at[b])
            return carry

        lax.fori_loop(0, n_groups, group, 0)

    return embed


def kernel(tokens, W_E):
    bsz, seq = tokens.shape
    b_total = bsz * seq
    idx = tokens.astype(jnp.int32).reshape(NW, b_total // NW // CHUNK, CHUNK)
    table = jnp.pad(W_E, ((0, 0), (0, D_PAD - D_MODEL)))
    out = _build(b_total)(idx, table)
    return out[:, :D_MODEL].reshape(bsz, seq, D_MODEL)
